# baseline (speedup 1.0000x reference)
import functools

import jax
import jax.numpy as jnp
from jax import lax
from jax.experimental import pallas as pl
from jax.experimental.pallas import tpu as pltpu
from jax.experimental.pallas import tpu_sc as plsc

N = 65536
NNZ = 4194304
NC = 2
NS = 16
L = 16
NW = NC * NS

CHUNK = 2048
TILE_ELEMS = NNZ // NW
NCHUNK = TILE_ELEMS // CHUNK
NBUF = 4
NVB = 2
ACC_SLICE = N // NS

_MESH = plsc.VectorSubcoreMesh(core_axis_name="c", subcore_axis_name="s")

_SCRATCH = (
    [pltpu.VMEM((N,), jnp.float32)]
    + [pltpu.VMEM((CHUNK,), jnp.int32)] * NBUF
    + [pltpu.VMEM((CHUNK,), jnp.int32)] * NBUF
    + [pltpu.VMEM((CHUNK,), jnp.float32)] * NBUF
    + [pltpu.VMEM((CHUNK,), jnp.float32)] * NVB
    + [pltpu.VMEM((CHUNK,), jnp.float32)] * NVB
    + [
        pltpu.VMEM((ACC_SLICE,), jnp.float32),
        pltpu.VMEM_SHARED((N,), jnp.float32),
        pltpu.SemaphoreType.DMA((NBUF,)),
        pltpu.SemaphoreType.DMA((NVB,)),
        pltpu.SemaphoreType.DMA,
    ]
)


@functools.partial(
    pl.kernel,
    out_type=jax.ShapeDtypeStruct((NC, N), jnp.float32),
    mesh=_MESH,
    compiler_params=pltpu.CompilerParams(needs_layout_passes=False),
    scratch_types=_SCRATCH,
)
def _sc_spmv(data_h, v_h, rows_h, cols_h, out_h, vbuf, *scratch):
    rbufs = scratch[0:NBUF]
    cbufs = scratch[NBUF:2 * NBUF]
    dbufs = scratch[2 * NBUF:3 * NBUF]
    avals = scratch[3 * NBUF:3 * NBUF + NVB]
    bvals = scratch[3 * NBUF + NVB:3 * NBUF + 2 * NVB]
    zbuf, acc, isem, ssem, vsem = scratch[3 * NBUF + 2 * NVB:]

    cid = lax.axis_index("c")
    sid = lax.axis_index("s")
    tile_chunk_base = (cid * NS + sid) * NCHUNK

    def issue_in(g, b):
        gi = tile_chunk_base + g
        pltpu.async_copy(rows_h.at[gi], rbufs[b], isem.at[b])
        pltpu.async_copy(cols_h.at[gi], cbufs[b], isem.at[b])
        pltpu.async_copy(data_h.at[gi], dbufs[b], isem.at[b])

    def wait_in(b):
        pltpu.make_async_copy(rows_h.at[0], rbufs[b], isem.at[b]).wait()
        pltpu.make_async_copy(cols_h.at[0], cbufs[b], isem.at[b]).wait()
        pltpu.make_async_copy(data_h.at[0], dbufs[b], isem.at[b]).wait()

    def drain_scatter(b2):
        pltpu.make_async_copy(data_h.at[0], avals[b2], ssem.at[b2]).wait()
        pltpu.make_async_copy(data_h.at[0], bvals[b2], ssem.at[b2]).wait()

    def compute(b, b2):
        @plsc.parallel_loop(0, CHUNK // L, unroll=1)
        def group_body(i):
            sl = pl.ds(i * L, L)
            r = rbufs[b][sl]
            c = cbufs[b][sl]
            d = dbufs[b][sl]
            vc = plsc.load_gather(vbuf, [c])
            vr = plsc.load_gather(vbuf, [r])
            avals[b2][sl] = d * vc
            bvals[b2][sl] = jnp.where(
                r != c, d * vr, jnp.zeros((L,), jnp.float32))

    def issue_scatter(b, b2):
        pltpu.async_copy(avals[b2], acc.at[rbufs[b]], ssem.at[b2], add=True)
        pltpu.async_copy(bvals[b2], acc.at[cbufs[b]], ssem.at[b2], add=True)

    vcp = pltpu.async_copy(v_h, vbuf, vsem)

    def zero_body(i, carry):
        zbuf[pl.ds(i * L, L)] = jnp.zeros((L,), jnp.float32)
        return carry
    lax.fori_loop(0, ACC_SLICE // L, zero_body, 0)
    pltpu.sync_copy(zbuf, acc.at[pl.ds(sid * ACC_SLICE, ACC_SLICE)])
    vcp.wait()
    plsc.subcore_barrier()

    issue_in(0, 0)
    issue_in(1, 1)

    def outer(t, carry):
        for bi in range(NBUF):
            g = t * NBUF + bi
            b2 = bi % NVB
            wait_in(bi)

            @pl.when(g >= NVB)
            def _():
                drain_scatter(b2)

            compute(bi, b2)
            issue_scatter(bi, b2)

            @pl.when(g + 2 < NCHUNK)
            def _():
                issue_in(g + 2, (bi + 2) % NBUF)
        return carry
    lax.fori_loop(0, NCHUNK // NBUF, outer, 0)

    drain_scatter(0)
    drain_scatter(1)
    plsc.subcore_barrier()

    pltpu.sync_copy(acc.at[pl.ds(sid * ACC_SLICE, ACC_SLICE)],
                    out_h.at[cid, pl.ds(sid * ACC_SLICE, ACC_SLICE)])


def _combine_body(p_ref, o_ref):
    o_ref[...] = p_ref[0] + p_ref[1]


def kernel(P_data, v, P_rows, P_cols):
    data2 = P_data.reshape(NNZ // CHUNK, CHUNK)
    rows2 = P_rows.reshape(NNZ // CHUNK, CHUNK)
    cols2 = P_cols.reshape(NNZ // CHUNK, CHUNK)
    parts = _sc_spmv(data2, v, rows2, cols2)
    p3 = parts.reshape(NC, N // 128, 128)
    out = pl.pallas_call(
        _combine_body,
        out_shape=jax.ShapeDtypeStruct((N // 128, 128), jnp.float32),
    )(p3)
    return out.reshape(N)


# speedup vs baseline: 1.4208x; 1.4208x over previous
import functools

import jax
import jax.numpy as jnp
from jax import lax
from jax.experimental import pallas as pl
from jax.experimental.pallas import tpu as pltpu
from jax.experimental.pallas import tpu_sc as plsc

N = 65536
NNZ = 4194304
NC = 2
NS = 16
L = 16
NW = NC * NS

ROW_W = 128
CHUNK_ROWS = 16
CHUNK = CHUNK_ROWS * ROW_W
TILE_ELEMS = NNZ // NW
NCHUNK = TILE_ELEMS // CHUNK
NBUF = 8
NDB = 4
NVB = 4
ACC_SLICE = N // NS

_MESH = plsc.VectorSubcoreMesh(core_axis_name="c", subcore_axis_name="s")

_SCRATCH = (
    [pltpu.VMEM((N,), jnp.float32)]
    + [pltpu.VMEM((CHUNK_ROWS, ROW_W), jnp.int32)] * NBUF
    + [pltpu.VMEM((CHUNK_ROWS, ROW_W), jnp.int32)] * NBUF
    + [pltpu.VMEM((CHUNK_ROWS, ROW_W), jnp.float32)] * NDB
    + [pltpu.VMEM((CHUNK_ROWS, ROW_W), jnp.float32)] * NVB
    + [pltpu.VMEM((CHUNK_ROWS, ROW_W), jnp.float32)] * NVB
    + [
        pltpu.VMEM((ACC_SLICE,), jnp.float32),
        pltpu.VMEM_SHARED((N,), jnp.float32),
        pltpu.SemaphoreType.DMA((NBUF,)),
        pltpu.SemaphoreType.DMA((NDB,)),
        pltpu.SemaphoreType.DMA((NVB,)),
        pltpu.SemaphoreType.DMA,
    ]
)


@functools.partial(
    pl.kernel,
    out_type=jax.ShapeDtypeStruct((NC, N), jnp.float32),
    mesh=_MESH,
    compiler_params=pltpu.CompilerParams(needs_layout_passes=False),
    scratch_types=_SCRATCH,
)
def _sc_spmv(data_h, v_h, rows_h, cols_h, out_h, vbuf, *scratch):
    rbufs = scratch[0:NBUF]
    cbufs = scratch[NBUF:2 * NBUF]
    dbufs = scratch[2 * NBUF:2 * NBUF + NDB]
    avals = scratch[2 * NBUF + NDB:2 * NBUF + NDB + NVB]
    bvals = scratch[2 * NBUF + NDB + NVB:2 * NBUF + NDB + 2 * NVB]
    zbuf, acc, isem, dsem, ssem, vsem = scratch[2 * NBUF + NDB + 2 * NVB:]

    cid = lax.axis_index("c")
    sid = lax.axis_index("s")
    tile_row_base = (cid * NS + sid) * (TILE_ELEMS // ROW_W)

    def issue_in(g, b, bd):
        rb = tile_row_base + g * CHUNK_ROWS
        pltpu.async_copy(rows_h.at[pl.ds(rb, CHUNK_ROWS), :], rbufs[b], isem.at[b])
        pltpu.async_copy(cols_h.at[pl.ds(rb, CHUNK_ROWS), :], cbufs[b], isem.at[b])
        pltpu.async_copy(data_h.at[pl.ds(rb, CHUNK_ROWS), :], dbufs[bd], dsem.at[bd])

    def wait_in(b, bd):
        pltpu.make_async_copy(rows_h.at[pl.ds(0, CHUNK_ROWS), :], rbufs[b], isem.at[b]).wait()
        pltpu.make_async_copy(cols_h.at[pl.ds(0, CHUNK_ROWS), :], cbufs[b], isem.at[b]).wait()
        pltpu.make_async_copy(data_h.at[pl.ds(0, CHUNK_ROWS), :], dbufs[bd], dsem.at[bd]).wait()

    def drain_scatter(b2):
        pltpu.make_async_copy(data_h.at[pl.ds(0, CHUNK_ROWS), :], avals[b2], ssem.at[b2]).wait()
        pltpu.make_async_copy(data_h.at[pl.ds(0, CHUNK_ROWS), :], bvals[b2], ssem.at[b2]).wait()

    def compute(b, bd, b2):
        @plsc.parallel_loop(0, CHUNK_ROWS, unroll=1)
        def row_body(i):
            for k in range(ROW_W // L):
                sl = pl.ds(k * L, L)
                r = rbufs[b][i, sl]
                c = cbufs[b][i, sl]
                d = dbufs[bd][i, sl]
                vc = plsc.load_gather(vbuf, [c])
                vr = plsc.load_gather(vbuf, [r])
                avals[b2][i, sl] = d * vc
                bvals[b2][i, sl] = jnp.where(
                    r != c, d * vr, jnp.zeros((L,), jnp.float32))

    def issue_scatter(b, b2):
        for i in range(CHUNK_ROWS):
            pltpu.async_copy(avals[b2].at[i], acc.at[rbufs[b].at[i]],
                             ssem.at[b2], add=True)
            pltpu.async_copy(bvals[b2].at[i], acc.at[cbufs[b].at[i]],
                             ssem.at[b2], add=True)

    vcp = pltpu.async_copy(v_h, vbuf, vsem)

    def zero_body(i, carry):
        zbuf[pl.ds(i * L, L)] = jnp.zeros((L,), jnp.float32)
        return carry
    lax.fori_loop(0, ACC_SLICE // L, zero_body, 0)
    pltpu.sync_copy(zbuf, acc.at[pl.ds(sid * ACC_SLICE, ACC_SLICE)])
    vcp.wait()
    plsc.subcore_barrier()

    issue_in(0, 0, 0)
    issue_in(1, 1, 1)

    def outer(t, carry):
        for bi in range(NBUF):
            g = t * NBUF + bi
            b2 = bi % NVB
            bd = bi % NDB
            wait_in(bi, bd)

            @pl.when(g >= NVB)
            def _():
                drain_scatter(b2)

            compute(bi, bd, b2)
            issue_scatter(bi, b2)

            @pl.when(g + 2 < NCHUNK)
            def _():
                issue_in(g + 2, (bi + 2) % NBUF, (bi + 2) % NDB)
        return carry
    lax.fori_loop(0, NCHUNK // NBUF, outer, 0)

    for b2 in range(NVB):
        drain_scatter(b2)
    plsc.subcore_barrier()

    pltpu.sync_copy(acc.at[pl.ds(sid * ACC_SLICE, ACC_SLICE)],
                    out_h.at[cid, pl.ds(sid * ACC_SLICE, ACC_SLICE)])


def _combine_body(p_ref, o_ref):
    o_ref[...] = p_ref[0] + p_ref[1]


def kernel(P_data, v, P_rows, P_cols):
    data2 = P_data.reshape(NNZ // ROW_W, ROW_W)
    rows2 = P_rows.reshape(NNZ // ROW_W, ROW_W)
    cols2 = P_cols.reshape(NNZ // ROW_W, ROW_W)
    parts = _sc_spmv(data2, v, rows2, cols2)
    p3 = parts.reshape(NC, N // ROW_W, ROW_W)
    out = pl.pallas_call(
        _combine_body,
        out_shape=jax.ShapeDtypeStruct((N // ROW_W, ROW_W), jnp.float32),
    )(p3)
    return out.reshape(N)


# speedup vs baseline: 1.4557x; 1.0245x over previous
import functools

import jax
import jax.numpy as jnp
from jax import lax
from jax.experimental import pallas as pl
from jax.experimental.pallas import tpu as pltpu
from jax.experimental.pallas import tpu_sc as plsc

N = 65536
NNZ = 4194304
NC = 2
NS = 16
L = 16
NW = NC * NS

ROW_W = 128
CHUNK_ROWS = 16
CHUNK = CHUNK_ROWS * ROW_W
TILE_ELEMS = NNZ // NW
NCHUNK = TILE_ELEMS // CHUNK
NBUF = 4
NDB = 4
NVB = 2
ACC_SLICE = N // NS

_MESH = plsc.VectorSubcoreMesh(core_axis_name="c", subcore_axis_name="s")

_SCRATCH = (
    [pltpu.VMEM((N,), jnp.float32)]
    + [pltpu.VMEM((CHUNK_ROWS, ROW_W), jnp.int32)] * NBUF
    + [pltpu.VMEM((CHUNK_ROWS, ROW_W), jnp.int32)] * NBUF
    + [pltpu.VMEM((CHUNK_ROWS, ROW_W), jnp.float32)] * NDB
    + [pltpu.VMEM((CHUNK_ROWS, ROW_W), jnp.float32)] * NVB
    + [pltpu.VMEM((CHUNK_ROWS, ROW_W), jnp.float32)] * NVB
    + [
        pltpu.VMEM((ACC_SLICE,), jnp.float32),
        pltpu.VMEM_SHARED((N,), jnp.float32),
        pltpu.SemaphoreType.DMA((NBUF,)),
        pltpu.SemaphoreType.DMA((NDB,)),
        pltpu.SemaphoreType.DMA((NVB,)),
        pltpu.SemaphoreType.DMA,
    ]
)


@functools.partial(
    pl.kernel,
    out_type=jax.ShapeDtypeStruct((NC, N), jnp.float32),
    mesh=_MESH,
    compiler_params=pltpu.CompilerParams(needs_layout_passes=False),
    scratch_types=_SCRATCH,
)
def _sc_spmv(data_h, v_h, rows_h, cols_h, out_h, vbuf, *scratch):
    rbufs = scratch[0:NBUF]
    cbufs = scratch[NBUF:2 * NBUF]
    dbufs = scratch[2 * NBUF:2 * NBUF + NDB]
    avals = scratch[2 * NBUF + NDB:2 * NBUF + NDB + NVB]
    bvals = scratch[2 * NBUF + NDB + NVB:2 * NBUF + NDB + 2 * NVB]
    zbuf, acc, isem, dsem, ssem, vsem = scratch[2 * NBUF + NDB + 2 * NVB:]

    cid = lax.axis_index("c")
    sid = lax.axis_index("s")
    tile_row_base = (cid * NS + sid) * (TILE_ELEMS // ROW_W)

    def issue_in(g, b, bd):
        rb = tile_row_base + g * CHUNK_ROWS
        pltpu.async_copy(rows_h.at[pl.ds(rb, CHUNK_ROWS), :], rbufs[b], isem.at[b])
        pltpu.async_copy(cols_h.at[pl.ds(rb, CHUNK_ROWS), :], cbufs[b], isem.at[b])
        pltpu.async_copy(data_h.at[pl.ds(rb, CHUNK_ROWS), :], dbufs[bd], dsem.at[bd])

    def wait_in(b, bd):
        pltpu.make_async_copy(rows_h.at[pl.ds(0, CHUNK_ROWS), :], rbufs[b], isem.at[b]).wait()
        pltpu.make_async_copy(cols_h.at[pl.ds(0, CHUNK_ROWS), :], cbufs[b], isem.at[b]).wait()
        pltpu.make_async_copy(data_h.at[pl.ds(0, CHUNK_ROWS), :], dbufs[bd], dsem.at[bd]).wait()

    def drain_scatter(b2):
        pltpu.make_async_copy(data_h.at[pl.ds(0, CHUNK_ROWS), :], avals[b2], ssem.at[b2]).wait()
        pltpu.make_async_copy(data_h.at[pl.ds(0, CHUNK_ROWS), :], bvals[b2], ssem.at[b2]).wait()

    def compute(b, bd, b2):
        @plsc.parallel_loop(0, CHUNK_ROWS, unroll=1)
        def row_body(i):
            for k in range(ROW_W // L):
                sl = pl.ds(k * L, L)
                r = rbufs[b][i, sl]
                c = cbufs[b][i, sl]
                d = dbufs[bd][i, sl]
                vc = plsc.load_gather(vbuf, [c])
                vr = plsc.load_gather(vbuf, [r])
                avals[b2][i, sl] = d * vc
                bvals[b2][i, sl] = jnp.where(
                    r != c, d * vr, jnp.zeros((L,), jnp.float32))

    def issue_scatter(b, b2):
        for i in range(CHUNK_ROWS):
            pltpu.async_copy(avals[b2].at[i], acc.at[rbufs[b].at[i]],
                             ssem.at[b2], add=True)
            pltpu.async_copy(bvals[b2].at[i], acc.at[cbufs[b].at[i]],
                             ssem.at[b2], add=True)

    vcp = pltpu.async_copy(v_h, vbuf, vsem)

    def zero_body(i, carry):
        zbuf[pl.ds(i * L, L)] = jnp.zeros((L,), jnp.float32)
        return carry
    lax.fori_loop(0, ACC_SLICE // L, zero_body, 0)
    pltpu.sync_copy(zbuf, acc.at[pl.ds(sid * ACC_SLICE, ACC_SLICE)])
    vcp.wait()
    plsc.subcore_barrier()

    issue_in(0, 0, 0)
    issue_in(1, 1, 1)

    def outer(t, carry):
        for bi in range(NBUF):
            g = t * NBUF + bi
            b2 = bi % NVB
            bd = bi % NDB
            wait_in(bi, bd)

            @pl.when(g >= NVB)
            def _():
                drain_scatter(b2)

            compute(bi, bd, b2)
            issue_scatter(bi, b2)

            @pl.when(g + 2 < NCHUNK)
            def _():
                issue_in(g + 2, (bi + 2) % NBUF, (bi + 2) % NDB)
        return carry
    lax.fori_loop(0, NCHUNK // NBUF, outer, 0)

    for b2 in range(NVB):
        drain_scatter(b2)
    plsc.subcore_barrier()

    pltpu.sync_copy(acc.at[pl.ds(sid * ACC_SLICE, ACC_SLICE)],
                    out_h.at[cid, pl.ds(sid * ACC_SLICE, ACC_SLICE)])


def _combine_body(p_ref, o_ref):
    o_ref[...] = p_ref[0] + p_ref[1]


def kernel(P_data, v, P_rows, P_cols):
    data2 = P_data.reshape(NNZ // ROW_W, ROW_W)
    rows2 = P_rows.reshape(NNZ // ROW_W, ROW_W)
    cols2 = P_cols.reshape(NNZ // ROW_W, ROW_W)
    parts = _sc_spmv(data2, v, rows2, cols2)
    p3 = parts.reshape(NC, N // ROW_W, ROW_W)
    out = pl.pallas_call(
        _combine_body,
        out_shape=jax.ShapeDtypeStruct((N // ROW_W, ROW_W), jnp.float32),
    )(p3)
    return out.reshape(N)
